# baseline (speedup 1.0000x reference)
import functools

import jax
import jax.numpy as jnp
from jax import lax
from jax.experimental import pallas as pl
from jax.experimental.pallas import tpu as pltpu
from jax.experimental.pallas import tpu_sc as plsc

E = 16
K = 2
D = 1024
H = 2048
T = 4096
CAP = 640

_PBITS = 13
_PMASK = (1 << _PBITS) - 1

_NW = 32
_ROWS = E * CAP
_RPT = _ROWS // _NW
_HB = 512
_NH = H // _HB


def _cumsum0(a):
    s = 1
    n = a.shape[0]
    while s < n:
        pad = jnp.zeros((s, a.shape[1]), a.dtype)
        a = a + jnp.concatenate([pad, a[:-s, :]], axis=0)
        s *= 2
    return a


def _router_body(x_ref, wg_ref, code_ref, wgt_ref):
    x = x_ref[...]
    wg = wg_ref[...]
    logits = jnp.dot(x, wg, preferred_element_type=jnp.float32)
    m = jnp.max(logits, axis=1, keepdims=True)
    ex = jnp.exp(logits - m)
    gates = ex / jnp.sum(ex, axis=1, keepdims=True)

    lane = lax.broadcasted_iota(jnp.int32, (T, E), 1)
    v0 = jnp.max(gates, axis=1, keepdims=True)
    i0 = jnp.min(jnp.where(gates == v0, lane, E), axis=1, keepdims=True)
    g1 = jnp.where(lane == i0, -1.0, gates)
    v1 = jnp.max(g1, axis=1, keepdims=True)
    i1 = jnp.min(jnp.where(g1 == v1, lane, E), axis=1, keepdims=True)

    denom = v0 + v1 + 1e-9
    w0 = v0 / denom
    w1 = v1 / denom

    oh0 = (lane == i0).astype(jnp.float32)
    oh1 = (lane == i1).astype(jnp.float32)
    c0 = _cumsum0(oh0)
    c1 = _cumsum0(oh1) + c0[T - 1:T, :]
    p0 = jnp.sum(oh0 * c0, axis=1, keepdims=True) - 1.0
    p1 = jnp.sum(oh1 * c1, axis=1, keepdims=True) - 1.0
    p0i = p0.astype(jnp.int32)
    p1i = p1.astype(jnp.int32)

    w0 = w0 * (p0i < CAP).astype(jnp.float32)
    w1 = w1 * (p1i < CAP).astype(jnp.float32)

    code_ref[...] = jnp.concatenate(
        [i0 * (1 << _PBITS) + p0i, i1 * (1 << _PBITS) + p1i], axis=1)
    wgt_ref[...] = jnp.concatenate([w0, w1], axis=1)


def _router(x, wg):
    return pl.pallas_call(
        _router_body,
        out_shape=[
            jax.ShapeDtypeStruct((T, K), jnp.int32),
            jax.ShapeDtypeStruct((T, K), jnp.float32),
        ],
    )(x, wg)


def _slotmap(code_f, wgt_f):
    mesh = plsc.VectorSubcoreMesh(core_axis_name="c", subcore_axis_name="s")

    @functools.partial(
        pl.kernel,
        mesh=mesh,
        compiler_params=pltpu.CompilerParams(needs_layout_passes=False),
        out_type=[
            jax.ShapeDtypeStruct((_ROWS,), jnp.int32),
            jax.ShapeDtypeStruct((_ROWS,), jnp.float32),
        ],
        scratch_types=[
            pltpu.VMEM((T * K,), jnp.int32),
            pltpu.VMEM((T * K,), jnp.float32),
            pltpu.VMEM((_ROWS,), jnp.int32),
            pltpu.VMEM((_ROWS,), jnp.float32),
        ],
    )
    def k(code_hbm, wgt_hbm, inv_hbm, ws_hbm, code_vm, wgt_vm, inv_vm, ws_vm):
        wid = lax.axis_index("s") * 2 + lax.axis_index("c")
        pltpu.sync_copy(code_hbm, code_vm)
        pltpu.sync_copy(wgt_hbm, wgt_vm)

        zero16i = jnp.zeros((16,), jnp.int32)
        zero16f = jnp.zeros((16,), jnp.float32)

        def zbody(i, carry):
            inv_vm[pl.ds(i * 16, 16)] = zero16i
            ws_vm[pl.ds(i * 16, 16)] = zero16f
            return carry

        lax.fori_loop(0, _ROWS // 16, zbody, 0, unroll=8)

        lanes = lax.iota(jnp.int32, 16)

        def sbody(kk, carry):
            t16 = kk * 16 + lanes
            f16 = t16 * K
            c0 = plsc.load_gather(code_vm, [f16])
            c1 = plsc.load_gather(code_vm, [f16 + 1])
            w0 = plsc.load_gather(wgt_vm, [f16])
            w1 = plsc.load_gather(wgt_vm, [f16 + 1])
            i0 = lax.shift_right_logical(c0, _PBITS)
            p0 = jnp.bitwise_and(c0, _PMASK)
            i1 = lax.shift_right_logical(c1, _PBITS)
            p1 = jnp.bitwise_and(c1, _PMASK)
            keep0 = p0 < CAP
            keep1 = p1 < CAP
            d0 = jnp.where(keep0, i0 * CAP + p0, 0)
            d1 = jnp.where(keep1, i1 * CAP + p1, 0)
            plsc.store_scatter(inv_vm, [d0], t16 + 1, mask=keep0)
            plsc.store_scatter(inv_vm, [d1], t16 + 1, mask=keep1)
            plsc.store_scatter(ws_vm, [d0], w0, mask=keep0)
            plsc.store_scatter(ws_vm, [d1], w1, mask=keep1)
            return carry

        lax.fori_loop(0, T // 16, sbody, 0, unroll=4)

        base = wid * _RPT
        pltpu.sync_copy(inv_vm.at[pl.ds(base, _RPT)], inv_hbm.at[pl.ds(base, _RPT)])
        pltpu.sync_copy(ws_vm.at[pl.ds(base, _RPT)], ws_hbm.at[pl.ds(base, _RPT)])

    return k(code_f, wgt_f)


def _moe_body(x_ref, inv_ref, ws_ref, w1_ref, b1_ref, w2_ref, b2_ref,
              out_ref, p_ref, disp_ref, yacc_ref):
    e = pl.program_id(0)
    h = pl.program_id(1)

    @pl.when(h == 0)
    def _build():
        tok = lax.broadcasted_iota(jnp.int32, (CAP, T), 1) + 1
        p_ref[...] = (tok == inv_ref[0]).astype(jnp.bfloat16)
        disp_ref[...] = jnp.dot(p_ref[...], x_ref[...],
                                preferred_element_type=jnp.float32
                                ).astype(jnp.bfloat16)
        yacc_ref[...] = jnp.zeros((CAP, D), jnp.float32)

    w1 = w1_ref[0].astype(jnp.bfloat16)
    hp = lax.dot_general(disp_ref[...], w1, (((1,), (1,)), ((), ())),
                         preferred_element_type=jnp.float32)
    hp = jnp.maximum(hp + b1_ref[0], 0.0)
    w2 = w2_ref[0].astype(jnp.bfloat16)
    yacc_ref[...] += lax.dot_general(hp.astype(jnp.bfloat16), w2,
                                     (((1,), (0,)), ((), ())),
                                     preferred_element_type=jnp.float32)

    @pl.when(h == _NH - 1)
    def _combine():
        y = yacc_ref[...] + b2_ref[0]
        ys = (y * ws_ref[0]).astype(jnp.bfloat16)
        for tch in range(2):
            blk = p_ref[:, pl.ds(tch * (T // 2), T // 2)]
            contrib = lax.dot_general(blk, ys, (((0,), (0,)), ((), ())),
                                      preferred_element_type=jnp.float32)

            @pl.when(e == 0)
            def _init():
                out_ref[pl.ds(tch * (T // 2), T // 2), :] = contrib

            @pl.when(e > 0)
            def _acc():
                out_ref[pl.ds(tch * (T // 2), T // 2), :] += contrib


def _moe(x_bf, inv, ws, fc1_w, fc1_b, fc2_w, fc2_b):
    return pl.pallas_call(
        _moe_body,
        grid=(E, _NH),
        in_specs=[
            pl.BlockSpec((T, D), lambda e, h: (0, 0)),
            pl.BlockSpec((1, CAP, 1), lambda e, h: (e, 0, 0)),
            pl.BlockSpec((1, CAP, 1), lambda e, h: (e, 0, 0)),
            pl.BlockSpec((1, _HB, D), lambda e, h: (e, h, 0)),
            pl.BlockSpec((1, 1, _HB), lambda e, h: (e, 0, h)),
            pl.BlockSpec((1, _HB, D), lambda e, h: (e, h, 0)),
            pl.BlockSpec((1, 1, D), lambda e, h: (e, 0, 0)),
        ],
        out_specs=pl.BlockSpec((T, D), lambda e, h: (0, 0)),
        out_shape=jax.ShapeDtypeStruct((T, D), jnp.float32),
        scratch_shapes=[
            pltpu.VMEM((CAP, T), jnp.bfloat16),
            pltpu.VMEM((CAP, D), jnp.bfloat16),
            pltpu.VMEM((CAP, D), jnp.float32),
        ],
    )(x_bf, inv, ws, fc1_w, fc1_b.reshape(E, 1, H), fc2_w,
      fc2_b.reshape(E, 1, D))


def kernel(x, wg, fc1_w, fc1_b, fc2_w, fc2_b):
    code, wgt = _router(x, wg)
    code_f = code.reshape(T * K)
    wgt_f = wgt.reshape(T * K)
    inv, ws = _slotmap(code_f, wgt_f)
    inv3 = inv.reshape(E, CAP, 1)
    ws3 = ws.reshape(E, CAP, 1)
    x_bf = x.astype(jnp.bfloat16)
    return _moe(x_bf, inv3, ws3, fc1_w, fc1_b, fc2_w, fc2_b)
